# Initial kernel scaffold; baseline (speedup 1.0000x reference)
import jax
import jax.numpy as jnp
from jax.experimental import pallas as pl


def kernel(indices, cu_seqlens, table):
    raise NotImplementedError("write your pallas kernel here")


# speedup vs baseline: 1.4959x; 1.4959x over previous
import functools

import jax
import jax.numpy as jnp
from jax import lax
from jax.experimental import pallas as pl
from jax.experimental.pallas import tpu as pltpu
from jax.experimental.pallas import tpu_sc as plsc

TOTAL_TOKENS = 16384
EMB = 512
_NC = 2
_NS = 16
_NW = _NC * _NS
_BPW = TOTAL_TOKENS // _NW
_CH = 128
_NCHUNK = _BPW // _CH


def _make_gather():
    mesh = plsc.VectorSubcoreMesh(core_axis_name="c", subcore_axis_name="s")

    @functools.partial(
        pl.kernel,
        mesh=mesh,
        out_type=jax.ShapeDtypeStruct((TOTAL_TOKENS, EMB), jnp.float32),
        scratch_types=[
            pltpu.VMEM((_NCHUNK, _CH), jnp.int32),
            pltpu.VMEM((_CH, EMB), jnp.float32),
            pltpu.SemaphoreType.DMA,
        ],
    )
    def gather_k(idx_hbm, table_hbm, out_hbm, idx_v, rows_v, sem):
        wid = lax.axis_index("s") * _NC + lax.axis_index("c")
        pltpu.sync_copy(idx_hbm.at[wid], idx_v)
        base = wid * _BPW
        for c in range(_NCHUNK):
            pltpu.async_copy(table_hbm.at[idx_v.at[c]], rows_v, sem).wait()
            pltpu.sync_copy(rows_v, out_hbm.at[pl.ds(base + c * _CH, _CH)])

    return gather_k


_gather = _make_gather()


def kernel(indices, cu_seqlens, table):
    del cu_seqlens
    idx = indices.astype(jnp.int32).reshape(_NW, _NCHUNK, _CH)
    return _gather(idx, table)
